# baseline (speedup 1.0000x reference)
import functools

import jax
import jax.numpy as jnp
from jax import lax
from jax.experimental import pallas as pl
from jax.experimental.pallas import tpu as pltpu
from jax.experimental.pallas import tpu_sc as plsc

BATCH = 4096
SEQ = 200
D = 32
VOCAB = 1000000
VROWS = VOCAB // 4
FLAT = BATCH * SEQ
NC = 2
NS = 16
NW = NC * NS
PER_W = FLAT // NW
CH = 400
NCHUNK = PER_W // CH
G = 80
NG = CH // G
GPR = G // 16
IROWS = PER_W // G
LANES = 16
SCALE = float(D) ** 0.5


def _emb_body(tok_hbm, q_hbm, tbl_hbm, pos_hbm, out_hbm,
              tok_v, q_v, rows_v, out_v, pos_v, sem):
    wid = lax.axis_index("s") * NC + lax.axis_index("c")
    base = wid * PER_W

    pltpu.sync_copy(pos_hbm, pos_v)

    iota = lax.iota(jnp.int32, LANES)

    def chunk_body(g, _):
        row0 = pl.multiple_of(base + g * CH, CH)

        pltpu.sync_copy(tok_hbm.at[wid * NCHUNK + g], tok_v)
        pltpu.sync_copy(q_hbm.at[wid * NCHUNK + g], q_v)

        copies = [
            pltpu.async_copy(
                tbl_hbm.at[q_v.at[a]],
                rows_v.at[pl.ds(a * G, G)],
                sem,
            )
            for a in range(NG)
        ]
        for c in copies:
            c.wait()

        for a in range(NG):

            def grp_body(b, _, a=a):
                sl = pl.ds(b * LANES, LANES)
                toks = tok_v[a, sl]
                offs = lax.shift_left(
                    lax.bitwise_and(toks, jnp.int32(3)), 5)
                j0 = a * G + b * LANES
                gi = a * GPR + b
                rows16 = j0 + iota
                for c in range(D):
                    val = plsc.load_gather(rows_v, [rows16, offs + c])
                    plsc.store_scatter(
                        out_v, [rows16, jnp.full((LANES,), c, jnp.int32)],
                        val + pos_v[gi, pl.ds(c * LANES, LANES)])
                return _

            lax.fori_loop(0, GPR, grp_body, None)

        pltpu.sync_copy(out_v, out_hbm.at[pl.ds(row0, CH)])
        return _

    lax.fori_loop(0, NCHUNK, chunk_body, None)


@jax.jit
def _emb_lookup(tok3d, q3d, tbl128, pos_table):
    mesh = plsc.VectorSubcoreMesh(core_axis_name="c", subcore_axis_name="s")
    fn = pl.kernel(
        _emb_body,
        mesh=mesh,
        compiler_params=pltpu.CompilerParams(needs_layout_passes=False),
        out_type=jax.ShapeDtypeStruct((FLAT, D), jnp.float32),
        scratch_types=[
            pltpu.VMEM((NG, G), jnp.int32),
            pltpu.VMEM((NG, G), jnp.int32),
            pltpu.VMEM((CH, 128), jnp.float32),
            pltpu.VMEM((CH, D), jnp.float32),
            pltpu.VMEM((CH // LANES, D * LANES), jnp.float32),
            pltpu.SemaphoreType.DMA,
        ],
    )
    return fn(tok3d, q3d, tbl128, pos_table)


def kernel(inputs, token_table, pos_table):
    flat = inputs.astype(jnp.int32).reshape(FLAT)
    tok3d = flat.reshape(NW * NCHUNK, NG, G)
    q3d = (flat >> 2).reshape(NW * NCHUNK, NG, G)
    tbl128 = (token_table * SCALE).reshape(VROWS, 128)
    p_idx = (LANES * jnp.arange(CH // LANES)[:, None]
             + jnp.arange(LANES)[None, :]) % SEQ
    pos_w = pos_table[p_idx, :].transpose(0, 2, 1).reshape(
        CH // LANES, D * LANES)
    out = _emb_lookup(tok3d, q3d, tbl128, pos_w)
    return out.reshape(BATCH, SEQ, D)


# speedup vs baseline: 1.0132x; 1.0132x over previous
import functools

import jax
import jax.numpy as jnp
from jax import lax
from jax.experimental import pallas as pl
from jax.experimental.pallas import tpu as pltpu
from jax.experimental.pallas import tpu_sc as plsc

BATCH = 4096
SEQ = 200
D = 32
VOCAB = 1000000
VROWS = VOCAB // 4
FLAT = BATCH * SEQ
NC = 2
NS = 16
NW = NC * NS
PER_W = FLAT // NW
CH = 400
NCHUNK = PER_W // CH
G = 80
NG = CH // G
GPR = G // 16
IROWS = PER_W // G
LANES = 16
SCALE = float(D) ** 0.5


def _emb_body(qt_hbm, tbl_hbm, pos_hbm, out_hbm,
              qt_v, rows_v, out_v, pos_v, sem):
    wid = lax.axis_index("s") * NC + lax.axis_index("c")
    base = wid * PER_W

    pltpu.sync_copy(pos_hbm, pos_v)

    iota = lax.iota(jnp.int32, LANES)

    def chunk_body(g, _):
        row0 = pl.multiple_of(base + g * CH, CH)

        pltpu.sync_copy(qt_hbm.at[wid * NCHUNK + g], qt_v)

        copies = [
            pltpu.async_copy(
                tbl_hbm.at[qt_v.at[a]],
                rows_v.at[pl.ds(a * G, G)],
                sem,
            )
            for a in range(NG)
        ]
        for c in copies:
            c.wait()

        for a in range(NG):

            def grp_body(b, _, a=a):
                sl = pl.ds(b * LANES, LANES)
                toks = qt_v[NG + a, sl]
                offs = lax.shift_left(
                    lax.bitwise_and(toks, jnp.int32(3)), 5)
                j0 = a * G + b * LANES
                gi = a * GPR + b
                rows16 = j0 + iota
                for c in range(D):
                    val = plsc.load_gather(rows_v, [rows16, offs + c])
                    plsc.store_scatter(
                        out_v, [rows16, jnp.full((LANES,), c, jnp.int32)],
                        val + pos_v[gi, pl.ds(c * LANES, LANES)])
                return _

            lax.fori_loop(0, GPR, grp_body, None)

        pltpu.sync_copy(out_v, out_hbm.at[pl.ds(row0, CH)])
        return _

    lax.fori_loop(0, NCHUNK, chunk_body, None)


@jax.jit
def _emb_lookup(qt3d, tbl128, pos_table):
    mesh = plsc.VectorSubcoreMesh(core_axis_name="c", subcore_axis_name="s")
    fn = pl.kernel(
        _emb_body,
        mesh=mesh,
        compiler_params=pltpu.CompilerParams(needs_layout_passes=False),
        out_type=jax.ShapeDtypeStruct((FLAT, D), jnp.float32),
        scratch_types=[
            pltpu.VMEM((2 * NG, G), jnp.int32),
            pltpu.VMEM((CH, 128), jnp.float32),
            pltpu.VMEM((CH, D), jnp.float32),
            pltpu.VMEM((CH // LANES, D * LANES), jnp.float32),
            pltpu.SemaphoreType.DMA,
        ],
    )
    return fn(qt3d, tbl128, pos_table)


def kernel(inputs, token_table, pos_table):
    flat = inputs.astype(jnp.int32).reshape(FLAT)
    tok3d = flat.reshape(NW * NCHUNK, NG, G)
    q3d = (flat >> 2).reshape(NW * NCHUNK, NG, G)
    qt3d = jnp.concatenate([q3d, tok3d], axis=1)
    tbl128 = (token_table * SCALE).reshape(VROWS, 128)
    p_idx = (LANES * jnp.arange(CH // LANES)[:, None]
             + jnp.arange(LANES)[None, :]) % SEQ
    pos_w = pos_table[p_idx, :].transpose(0, 2, 1).reshape(
        CH // LANES, D * LANES)
    out = _emb_lookup(qt3d, tbl128, pos_w)
    return out.reshape(BATCH, SEQ, D)
